# baseline (speedup 1.0000x reference)
import functools
import jax
import jax.numpy as jnp
from jax.experimental import pallas as pl
from jax.experimental.pallas import tpu as pltpu

_LANE = 128


def _moe_body(n_routed, n_inter_chunks, x_ref, w1_ref, w2_ref, rwt_ref,
              out_ref, c_ref):
    e = pl.program_id(1)
    ib = pl.program_id(2)
    xb = x_ref[...]

    @pl.when(ib == 0)
    def _compute_combined_weight():
        logits = jnp.dot(xb, rwt_ref[...], preferred_element_type=jnp.float32)
        lane = jax.lax.broadcasted_iota(jnp.int32, logits.shape, 1)
        neg = jnp.float32(-1e30)
        l = jnp.where(lane < n_routed, logits, neg)
        m0 = jnp.max(l, axis=1, keepdims=True)
        i0 = jnp.min(jnp.where(l == m0, lane, 9999), axis=1, keepdims=True)
        l2 = jnp.where(lane == i0, neg, l)
        m1 = jnp.max(l2, axis=1, keepdims=True)
        i1 = jnp.min(jnp.where(l2 == m1, lane, 9999), axis=1, keepdims=True)
        c0 = jax.nn.sigmoid(m0 - m1)
        c = jnp.where(i0 == e, c0, 0.0) + jnp.where(i1 == e, 1.0 - c0, 0.0)
        c = jnp.where(e >= n_routed, 1.0, c)
        c_ref[...] = jnp.broadcast_to(c, c_ref.shape)

    w1c = w1_ref[0]
    w2c = w2_ref[0]
    h = jax.lax.dot_general(xb, w1c, (((1,), (1,)), ((), ())),
                            preferred_element_type=jnp.float32)
    h = h * jax.nn.sigmoid(h)
    y = jax.lax.dot_general(h, w2c, (((1,), (1,)), ((), ())),
                            preferred_element_type=jnp.float32)
    y = y * c_ref[:, 0:1]

    @pl.when((e == 0) & (ib == 0))
    def _init():
        out_ref[...] = y

    @pl.when((e > 0) | (ib > 0))
    def _acc():
        out_ref[...] += y


def kernel(hidden_states, shared_w1, shared_w2, routed_w1, routed_w2,
           router_w):
    bsz, seq, hdim = hidden_states.shape
    T = bsz * seq
    n_routed, inter, _ = routed_w1.shape
    n_shared = shared_w1.shape[0]
    n_exp = n_routed + n_shared

    x = hidden_states.reshape(T, hdim)
    w1 = jnp.concatenate([routed_w1, shared_w1], axis=0)
    w2 = jnp.concatenate([routed_w2, shared_w2], axis=0)
    rwt = jnp.zeros((hdim, _LANE), jnp.float32).at[:, :n_routed].set(
        router_w.T)

    TB = 512 if T % 512 == 0 else T
    IB = 1024 if inter % 1024 == 0 else inter
    n_tb = T // TB
    n_ib = inter // IB

    body = functools.partial(_moe_body, n_routed, n_ib)
    out = pl.pallas_call(
        body,
        grid=(n_tb, n_exp, n_ib),
        in_specs=[
            pl.BlockSpec((TB, hdim), lambda t, e, ib: (t, 0)),
            pl.BlockSpec((1, IB, hdim), lambda t, e, ib: (e, ib, 0)),
            pl.BlockSpec((1, hdim, IB), lambda t, e, ib: (e, 0, ib)),
            pl.BlockSpec((hdim, _LANE), lambda t, e, ib: (0, 0)),
        ],
        out_specs=pl.BlockSpec((TB, hdim), lambda t, e, ib: (t, 0)),
        out_shape=jax.ShapeDtypeStruct((T, hdim), jnp.float32),
        scratch_shapes=[pltpu.VMEM((TB, _LANE), jnp.float32)],
        compiler_params=pltpu.CompilerParams(
            dimension_semantics=("parallel", "arbitrary", "arbitrary")),
    )(x, w1, w2, rwt)
    return out.reshape(bsz, seq, hdim)


# speedup vs baseline: 1.1567x; 1.1567x over previous
import functools
import jax
import jax.numpy as jnp
from jax.experimental import pallas as pl
from jax.experimental.pallas import tpu as pltpu

_LANE = 128


def _moe_body(n_routed, n_inter_chunks, x_ref, w1_ref, w2_ref, rwt_ref,
              out_ref, c_ref):
    e = pl.program_id(1)
    ib = pl.program_id(2)
    xb = x_ref[...]

    @pl.when(ib == 0)
    def _compute_combined_weight():
        logits = jnp.dot(xb, rwt_ref[...], preferred_element_type=jnp.float32)
        lane = jax.lax.broadcasted_iota(jnp.int32, logits.shape, 1)
        neg = jnp.float32(-1e30)
        l = jnp.where(lane < n_routed, logits, neg)
        m0 = jnp.max(l, axis=1, keepdims=True)
        i0 = jnp.min(jnp.where(l == m0, lane, 9999), axis=1, keepdims=True)
        l2 = jnp.where(lane == i0, neg, l)
        m1 = jnp.max(l2, axis=1, keepdims=True)
        i1 = jnp.min(jnp.where(l2 == m1, lane, 9999), axis=1, keepdims=True)
        c0 = jax.nn.sigmoid(m0 - m1)
        c = jnp.where(i0 == e, c0, 0.0) + jnp.where(i1 == e, 1.0 - c0, 0.0)
        c = jnp.where(e >= n_routed, 1.0, c)
        c_ref[...] = jnp.broadcast_to(c, c_ref.shape)

    w1c = w1_ref[0]
    w2c = w2_ref[0]
    h = jax.lax.dot_general(xb, w1c, (((1,), (1,)), ((), ())),
                            preferred_element_type=jnp.float32)
    h = h * jax.nn.sigmoid(h)
    y = jax.lax.dot_general(h, w2c, (((1,), (1,)), ((), ())),
                            preferred_element_type=jnp.float32)
    y = y * c_ref[:, 0:1]

    @pl.when((e == 0) & (ib == 0))
    def _init():
        out_ref[...] = y

    @pl.when((e > 0) | (ib > 0))
    def _acc():
        out_ref[...] += y


def kernel(hidden_states, shared_w1, shared_w2, routed_w1, routed_w2,
           router_w):
    bsz, seq, hdim = hidden_states.shape
    T = bsz * seq
    n_routed, inter, _ = routed_w1.shape
    n_shared = shared_w1.shape[0]
    n_exp = n_routed + n_shared

    x = hidden_states.reshape(T, hdim)
    w1 = jnp.concatenate([routed_w1, shared_w1], axis=0)
    w2 = jnp.concatenate([routed_w2, shared_w2], axis=0)
    rwt = jnp.zeros((hdim, _LANE), jnp.float32).at[:, :n_routed].set(
        router_w.T)

    TB = 1024 if T % 1024 == 0 else T
    IB = 1024 if inter % 1024 == 0 else inter
    n_tb = T // TB
    n_ib = inter // IB

    body = functools.partial(_moe_body, n_routed, n_ib)
    out = pl.pallas_call(
        body,
        grid=(n_tb, n_exp, n_ib),
        in_specs=[
            pl.BlockSpec((TB, hdim), lambda t, e, ib: (t, 0)),
            pl.BlockSpec((1, IB, hdim), lambda t, e, ib: (e, ib, 0)),
            pl.BlockSpec((1, hdim, IB), lambda t, e, ib: (e, 0, ib)),
            pl.BlockSpec((hdim, _LANE), lambda t, e, ib: (0, 0)),
        ],
        out_specs=pl.BlockSpec((TB, hdim), lambda t, e, ib: (t, 0)),
        out_shape=jax.ShapeDtypeStruct((T, hdim), jnp.float32),
        scratch_shapes=[pltpu.VMEM((TB, _LANE), jnp.float32)],
        compiler_params=pltpu.CompilerParams(
            dimension_semantics=("parallel", "arbitrary", "arbitrary")),
    )(x, w1, w2, rwt)
    return out.reshape(bsz, seq, hdim)
